# baseline (device time: 61268 ns/iter reference)
import jax
import jax.numpy as jnp
from jax import lax
from jax.experimental import pallas as pl
from jax.experimental.pallas import tpu as pltpu


def kernel(x, W):
    t, d = x.shape
    _, v_loc = W.shape
    v_glob = 2 * v_loc

    def body(x_ref, w_ref, out_ref, send_buf, recv_buf, send_sem, recv_sem):
        my_x = lax.axis_index("x")
        my_y = lax.axis_index("y")
        my_z = lax.axis_index("z")
        peer = (1 - my_x, my_y, my_z)

        barrier = pltpu.get_barrier_semaphore()
        pl.semaphore_signal(
            barrier, inc=1, device_id=peer, device_id_type=pl.DeviceIdType.MESH
        )
        pl.semaphore_wait(barrier, 1)

        send_buf[:, :] = jnp.dot(
            x_ref[:, :], w_ref[:, :], preferred_element_type=jnp.float32
        )

        rdma = pltpu.make_async_remote_copy(
            src_ref=send_buf,
            dst_ref=recv_buf,
            send_sem=send_sem,
            recv_sem=recv_sem,
            device_id=peer,
            device_id_type=pl.DeviceIdType.MESH,
        )
        rdma.start()
        rdma.wait()

        mine = send_buf[:, :]
        theirs = recv_buf[:, :]
        m = jnp.maximum(
            jnp.max(mine, axis=-1, keepdims=True),
            jnp.max(theirs, axis=-1, keepdims=True),
        )
        e_mine = jnp.exp(mine - m)
        e_theirs = jnp.exp(theirs - m)
        s = jnp.sum(e_mine, axis=-1, keepdims=True) + jnp.sum(
            e_theirs, axis=-1, keepdims=True
        )
        p_mine = e_mine / s
        p_theirs = e_theirs / s

        @pl.when(my_x == 0)
        def _():
            out_ref[:, :v_loc] = p_mine
            out_ref[:, v_loc:] = p_theirs

        @pl.when(my_x == 1)
        def _():
            out_ref[:, :v_loc] = p_theirs
            out_ref[:, v_loc:] = p_mine

    return pl.pallas_call(
        body,
        out_shape=jax.ShapeDtypeStruct((t, v_glob), jnp.float32),
        in_specs=[
            pl.BlockSpec(memory_space=pltpu.VMEM),
            pl.BlockSpec(memory_space=pltpu.VMEM),
        ],
        out_specs=pl.BlockSpec(memory_space=pltpu.VMEM),
        scratch_shapes=[
            pltpu.VMEM((t, v_loc), jnp.float32),
            pltpu.VMEM((t, v_loc), jnp.float32),
            pltpu.SemaphoreType.DMA,
            pltpu.SemaphoreType.DMA,
        ],
        compiler_params=pltpu.CompilerParams(collective_id=0),
    )(x, W)


# device time: 57910 ns/iter; 1.0580x vs baseline; 1.0580x over previous
import jax
import jax.numpy as jnp
from jax import lax
from jax.experimental import pallas as pl
from jax.experimental.pallas import tpu as pltpu

N_CHUNKS = 8


def kernel(x, W):
    t, d = x.shape
    _, v_loc = W.shape
    v_glob = 2 * v_loc
    vc = v_loc // N_CHUNKS

    def body(x_ref, w_ref, out_ref, send_buf, recv_buf, send_sems, recv_sems):
        my_x = lax.axis_index("x")
        my_y = lax.axis_index("y")
        my_z = lax.axis_index("z")
        peer = (1 - my_x, my_y, my_z)

        barrier = pltpu.get_barrier_semaphore()
        pl.semaphore_signal(
            barrier, inc=1, device_id=peer, device_id_type=pl.DeviceIdType.MESH
        )
        pl.semaphore_wait(barrier, 1)

        rdmas = []
        xv = x_ref[:, :]
        for i in range(N_CHUNKS):
            send_buf[i] = jnp.dot(
                xv, w_ref[:, i * vc:(i + 1) * vc],
                preferred_element_type=jnp.float32,
            )
            rdma = pltpu.make_async_remote_copy(
                src_ref=send_buf.at[i],
                dst_ref=recv_buf.at[i],
                send_sem=send_sems.at[i],
                recv_sem=recv_sems.at[i],
                device_id=peer,
                device_id_type=pl.DeviceIdType.MESH,
            )
            rdma.start()
            rdmas.append(rdma)

        s = jnp.zeros((t, 1), jnp.float32)
        for i in range(N_CHUNKS):
            e = jnp.exp(send_buf[i])
            s = s + jnp.sum(e, axis=-1, keepdims=True)

            @pl.when(my_x == 0)
            def _():
                out_ref[:, i * vc:(i + 1) * vc] = e

            @pl.when(my_x == 1)
            def _():
                out_ref[:, v_loc + i * vc:v_loc + (i + 1) * vc] = e

        for i in range(N_CHUNKS):
            rdmas[i].wait_recv()
            e = jnp.exp(recv_buf[i])
            s = s + jnp.sum(e, axis=-1, keepdims=True)

            @pl.when(my_x == 0)
            def _():
                out_ref[:, v_loc + i * vc:v_loc + (i + 1) * vc] = e

            @pl.when(my_x == 1)
            def _():
                out_ref[:, i * vc:(i + 1) * vc] = e

        out_ref[:, :] = out_ref[:, :] * (1.0 / s)

        for i in range(N_CHUNKS):
            rdmas[i].wait_send()

    return pl.pallas_call(
        body,
        out_shape=jax.ShapeDtypeStruct((t, v_glob), jnp.float32),
        in_specs=[
            pl.BlockSpec(memory_space=pltpu.VMEM),
            pl.BlockSpec(memory_space=pltpu.VMEM),
        ],
        out_specs=pl.BlockSpec(memory_space=pltpu.VMEM),
        scratch_shapes=[
            pltpu.VMEM((N_CHUNKS, t, vc), jnp.float32),
            pltpu.VMEM((N_CHUNKS, t, vc), jnp.float32),
            pltpu.SemaphoreType.DMA((N_CHUNKS,)),
            pltpu.SemaphoreType.DMA((N_CHUNKS,)),
        ],
        compiler_params=pltpu.CompilerParams(collective_id=0),
    )(x, W)


# device time: 41105 ns/iter; 1.4905x vs baseline; 1.4088x over previous
import jax
import jax.numpy as jnp
from jax import lax
from jax.experimental import pallas as pl
from jax.experimental.pallas import tpu as pltpu

PARTS = 4
SUB = 4
MESH = pl.DeviceIdType.MESH


def kernel(x, W):
    t, d = x.shape
    _, v_loc = W.shape
    v_glob = 2 * v_loc
    part_cols = v_loc // PARTS
    ch = part_cols // SUB

    def body(x_ref, w_ref, out_ref, lbuf, rbuf, xs_sems, ys_sems, zs_sems,
             recv_sems):
        mx = lax.axis_index("x")
        my = lax.axis_index("y")
        mz = lax.axis_index("z")
        xpeer = (1 - mx, my, mz)
        ypeer = (mx, 1 - my, mz)
        zpeer = (mx, my, 1 - mz)
        pm = 2 * my + mz
        pyn = 2 * (1 - my) + mz
        pzn = 2 * my + (1 - mz)
        pd = 2 * (1 - my) + (1 - mz)

        barrier = pltpu.get_barrier_semaphore()
        for nbr in (xpeer, ypeer, zpeer):
            pl.semaphore_signal(barrier, inc=1, device_id=nbr,
                                device_id_type=MESH)
        pl.semaphore_wait(barrier, 3)

        xv = x_ref[:, :]
        for p in range(PARTS):
            for c in range(SUB):
                col = p * part_cols + c * ch
                lbuf[p, c] = jnp.dot(
                    xv, w_ref[:, col:col + ch],
                    preferred_element_type=jnp.float32,
                )

                @pl.when(pm == p)
                def _(p=p, c=c):
                    r = pltpu.make_async_remote_copy(
                        src_ref=lbuf.at[p, c], dst_ref=rbuf.at[p, c],
                        send_sem=xs_sems.at[c], recv_sem=recv_sems.at[p, c],
                        device_id=xpeer, device_id_type=MESH,
                    )
                    r.start()

        s = jnp.zeros((t, 1), jnp.float32)
        for p in range(PARTS):
            for c in range(SUB):
                e = jnp.exp(lbuf[p, c])
                s = s + jnp.sum(e, axis=-1, keepdims=True)
                col = p * part_cols + c * ch

                @pl.when(mx == 0)
                def _(col=col, e=e):
                    out_ref[:, col:col + ch] = e

                @pl.when(mx == 1)
                def _(col=col, e=e):
                    out_ref[:, v_loc + col:v_loc + col + ch] = e

        def recv_wait(p, c):
            rd = pltpu.make_async_remote_copy(
                src_ref=rbuf.at[p, c], dst_ref=rbuf.at[p, c],
                send_sem=xs_sems.at[0],
                recv_sem=recv_sems.at[p, c],
                device_id=xpeer, device_id_type=MESH,
            )
            rd.wait_recv()

        def fwd(p, c, sems, k, peer):
            r = pltpu.make_async_remote_copy(
                src_ref=rbuf.at[p, c], dst_ref=rbuf.at[p, c],
                send_sem=sems.at[k], recv_sem=recv_sems.at[p, c],
                device_id=peer, device_id_type=MESH,
            )
            return r

        order = [pm, pyn, pzn, pd]
        yk = 0
        zk = 0
        for j in range(PARTS):
            p = order[j]
            for c in range(SUB):
                recv_wait(p, c)
                if j == 0:
                    fwd(p, c, ys_sems, yk, ypeer).start()
                    yk += 1
                    fwd(p, c, zs_sems, zk, zpeer).start()
                    zk += 1
                elif j == 1 and c >= SUB // 2:
                    fwd(p, c, zs_sems, zk, zpeer).start()
                    zk += 1
                elif j == 2 and c < SUB // 2:
                    fwd(p, c, ys_sems, yk, ypeer).start()
                    yk += 1

        for p in range(PARTS):
            for c in range(SUB):
                e = jnp.exp(rbuf[p, c])
                s = s + jnp.sum(e, axis=-1, keepdims=True)
                col = p * part_cols + c * ch

                @pl.when(mx == 0)
                def _(col=col, e=e):
                    out_ref[:, v_loc + col:v_loc + col + ch] = e

                @pl.when(mx == 1)
                def _(col=col, e=e):
                    out_ref[:, col:col + ch] = e

        out_ref[:, :] = out_ref[:, :] * (1.0 / s)

        for p in range(PARTS):
            for c in range(SUB):
                @pl.when(pm == p)
                def _(p=p, c=c):
                    r = pltpu.make_async_remote_copy(
                        src_ref=lbuf.at[p, c], dst_ref=rbuf.at[p, c],
                        send_sem=xs_sems.at[c], recv_sem=recv_sems.at[p, c],
                        device_id=xpeer, device_id_type=MESH,
                    )
                    r.wait_send()
        yk = 0
        zk = 0
        for c in range(SUB):
            fwd(pm, c, ys_sems, yk, ypeer).wait_send()
            yk += 1
            fwd(pm, c, zs_sems, zk, zpeer).wait_send()
            zk += 1
        for c in range(SUB // 2, SUB):
            fwd(pyn, c, zs_sems, zk, zpeer).wait_send()
            zk += 1
        for c in range(SUB // 2):
            fwd(pzn, c, ys_sems, yk, ypeer).wait_send()
            yk += 1

    n_face_sends = SUB + SUB // 2
    return pl.pallas_call(
        body,
        out_shape=jax.ShapeDtypeStruct((t, v_glob), jnp.float32),
        in_specs=[
            pl.BlockSpec(memory_space=pltpu.VMEM),
            pl.BlockSpec(memory_space=pltpu.VMEM),
        ],
        out_specs=pl.BlockSpec(memory_space=pltpu.VMEM),
        scratch_shapes=[
            pltpu.VMEM((PARTS, SUB, t, ch), jnp.float32),
            pltpu.VMEM((PARTS, SUB, t, ch), jnp.float32),
            pltpu.SemaphoreType.DMA((SUB,)),
            pltpu.SemaphoreType.DMA((n_face_sends,)),
            pltpu.SemaphoreType.DMA((n_face_sends,)),
            pltpu.SemaphoreType.DMA((PARTS, SUB)),
        ],
        compiler_params=pltpu.CompilerParams(collective_id=0),
    )(x, W)


# device time: 39403 ns/iter; 1.5549x vs baseline; 1.0432x over previous
import jax
import jax.numpy as jnp
from jax import lax
from jax.experimental import pallas as pl
from jax.experimental.pallas import tpu as pltpu

PARTS = 4
SUB = 4
MESH = pl.DeviceIdType.MESH


def kernel(x, W):
    t, d = x.shape
    _, v_loc = W.shape
    v_glob = 2 * v_loc
    part_cols = v_loc // PARTS
    ch = part_cols // SUB

    def body(x_ref, w_ref, out_ref, lbuf, rbuf, xs_sems, ys_sems, zs_sems,
             recv_sems):
        mx = lax.axis_index("x")
        my = lax.axis_index("y")
        mz = lax.axis_index("z")
        xpeer = (1 - mx, my, mz)
        ypeer = (mx, 1 - my, mz)
        zpeer = (mx, my, 1 - mz)
        pm = 2 * my + mz
        pyn = 2 * (1 - my) + mz
        pzn = 2 * my + (1 - mz)
        pd = 2 * (1 - my) + (1 - mz)

        barrier = pltpu.get_barrier_semaphore()
        for nbr in (xpeer, ypeer, zpeer):
            pl.semaphore_signal(barrier, inc=1, device_id=nbr,
                                device_id_type=MESH)
        pl.semaphore_wait(barrier, 3)

        xv = x_ref[:, :]
        for j in range(PARTS):
            p = lax.rem(pm + j, PARTS)
            for c in range(SUB):
                col = p * part_cols + c * ch
                lbuf[p, c] = jnp.dot(
                    xv, w_ref[:, pl.ds(col, ch)],
                    preferred_element_type=jnp.float32,
                )
                if j == 0:
                    r = pltpu.make_async_remote_copy(
                        src_ref=lbuf.at[p, c], dst_ref=rbuf.at[p, c],
                        send_sem=xs_sems.at[c], recv_sem=recv_sems.at[p, c],
                        device_id=xpeer, device_id_type=MESH,
                    )
                    r.start()

        s = jnp.zeros((t, 1), jnp.float32)
        for p in range(PARTS):
            for c in range(SUB):
                e = jnp.exp(lbuf[p, c])
                s = s + jnp.sum(e, axis=-1, keepdims=True)
                col = p * part_cols + c * ch

                @pl.when(mx == 0)
                def _(col=col, e=e):
                    out_ref[:, col:col + ch] = e

                @pl.when(mx == 1)
                def _(col=col, e=e):
                    out_ref[:, v_loc + col:v_loc + col + ch] = e

        def recv_wait(p, c):
            rd = pltpu.make_async_remote_copy(
                src_ref=rbuf.at[p, c], dst_ref=rbuf.at[p, c],
                send_sem=xs_sems.at[0],
                recv_sem=recv_sems.at[p, c],
                device_id=xpeer, device_id_type=MESH,
            )
            rd.wait_recv()

        def fwd(p, c, sems, k, peer):
            r = pltpu.make_async_remote_copy(
                src_ref=rbuf.at[p, c], dst_ref=rbuf.at[p, c],
                send_sem=sems.at[k], recv_sem=recv_sems.at[p, c],
                device_id=peer, device_id_type=MESH,
            )
            return r

        order = [pm, pyn, pzn, pd]
        yk = 0
        zk = 0
        for j in range(PARTS):
            p = order[j]
            for c in range(SUB):
                recv_wait(p, c)
                if j == 0:
                    fwd(p, c, ys_sems, yk, ypeer).start()
                    yk += 1
                    fwd(p, c, zs_sems, zk, zpeer).start()
                    zk += 1
                elif j == 1 and c >= SUB // 2:
                    fwd(p, c, zs_sems, zk, zpeer).start()
                    zk += 1
                elif j == 2 and c < SUB // 2:
                    fwd(p, c, ys_sems, yk, ypeer).start()
                    yk += 1

        for p in range(PARTS):
            for c in range(SUB):
                e = jnp.exp(rbuf[p, c])
                s = s + jnp.sum(e, axis=-1, keepdims=True)
                col = p * part_cols + c * ch

                @pl.when(mx == 0)
                def _(col=col, e=e):
                    out_ref[:, v_loc + col:v_loc + col + ch] = e

                @pl.when(mx == 1)
                def _(col=col, e=e):
                    out_ref[:, col:col + ch] = e

        out_ref[:, :] = out_ref[:, :] * (1.0 / s)

        for c in range(SUB):
            r = pltpu.make_async_remote_copy(
                src_ref=lbuf.at[pm, c], dst_ref=rbuf.at[pm, c],
                send_sem=xs_sems.at[c], recv_sem=recv_sems.at[pm, c],
                device_id=xpeer, device_id_type=MESH,
            )
            r.wait_send()
        yk = 0
        zk = 0
        for c in range(SUB):
            fwd(pm, c, ys_sems, yk, ypeer).wait_send()
            yk += 1
            fwd(pm, c, zs_sems, zk, zpeer).wait_send()
            zk += 1
        for c in range(SUB // 2, SUB):
            fwd(pyn, c, zs_sems, zk, zpeer).wait_send()
            zk += 1
        for c in range(SUB // 2):
            fwd(pzn, c, ys_sems, yk, ypeer).wait_send()
            yk += 1

    n_face_sends = SUB + SUB // 2
    return pl.pallas_call(
        body,
        out_shape=jax.ShapeDtypeStruct((t, v_glob), jnp.float32),
        in_specs=[
            pl.BlockSpec(memory_space=pltpu.VMEM),
            pl.BlockSpec(memory_space=pltpu.VMEM),
        ],
        out_specs=pl.BlockSpec(memory_space=pltpu.VMEM),
        scratch_shapes=[
            pltpu.VMEM((PARTS, SUB, t, ch), jnp.float32),
            pltpu.VMEM((PARTS, SUB, t, ch), jnp.float32),
            pltpu.SemaphoreType.DMA((SUB,)),
            pltpu.SemaphoreType.DMA((n_face_sends,)),
            pltpu.SemaphoreType.DMA((n_face_sends,)),
            pltpu.SemaphoreType.DMA((PARTS, SUB)),
        ],
        compiler_params=pltpu.CompilerParams(collective_id=0),
    )(x, W)


# device time: 37369 ns/iter; 1.6395x vs baseline; 1.0544x over previous
import jax
import jax.numpy as jnp
from jax import lax
from jax.experimental import pallas as pl
from jax.experimental.pallas import tpu as pltpu

PARTS = 4
SUB = 8
XD = 3
MESH = pl.DeviceIdType.MESH


def kernel(x, W):
    t, d = x.shape
    _, v_loc = W.shape
    v_glob = 2 * v_loc
    part_cols = v_loc // PARTS
    ch = part_cols // SUB
    yd = (SUB - XD + 1) // 2
    zd = SUB - XD - yd

    def body(x_ref, w_ref, out_ref, lbuf, rbuf, xs_sems, xd_sems, ys_sems,
             zs_sems, recv_sems):
        mx = lax.axis_index("x")
        my = lax.axis_index("y")
        mz = lax.axis_index("z")
        xpeer = (1 - mx, my, mz)
        ypeer = (mx, 1 - my, mz)
        zpeer = (mx, my, 1 - mz)
        pm = 2 * my + mz
        pyn = 2 * (1 - my) + mz
        pzn = 2 * my + (1 - mz)
        pd = 2 * (1 - my) + (1 - mz)

        barrier = pltpu.get_barrier_semaphore()
        for nbr in (xpeer, ypeer, zpeer):
            pl.semaphore_signal(barrier, inc=1, device_id=nbr,
                                device_id_type=MESH)
        pl.semaphore_wait(barrier, 3)

        xv = x_ref[:, :]
        for j in range(PARTS):
            p = lax.rem(pm + j, PARTS)
            for c in range(SUB):
                col = p * part_cols + c * ch
                lbuf[p, c] = jnp.dot(
                    xv, w_ref[:, pl.ds(col, ch)],
                    preferred_element_type=jnp.float32,
                )
                if j == 0:
                    r = pltpu.make_async_remote_copy(
                        src_ref=lbuf.at[p, c], dst_ref=rbuf.at[p, c],
                        send_sem=xs_sems.at[c], recv_sem=recv_sems.at[p, c],
                        device_id=xpeer, device_id_type=MESH,
                    )
                    r.start()

        for c in range(XD):
            r = pltpu.make_async_remote_copy(
                src_ref=lbuf.at[pd, c], dst_ref=rbuf.at[pd, c],
                send_sem=xd_sems.at[c], recv_sem=recv_sems.at[pd, c],
                device_id=xpeer, device_id_type=MESH,
            )
            r.start()

        s = jnp.zeros((t, 1), jnp.float32)
        lbase = mx * v_loc
        for p in range(PARTS):
            for c in range(SUB):
                e = jnp.exp(lbuf[p, c])
                s = s + jnp.sum(e, axis=-1, keepdims=True)
                out_ref[:, pl.ds(lbase + p * part_cols + c * ch, ch)] = e

        def recv_wait(p, c):
            rd = pltpu.make_async_remote_copy(
                src_ref=rbuf.at[p, c], dst_ref=rbuf.at[p, c],
                send_sem=xs_sems.at[0],
                recv_sem=recv_sems.at[p, c],
                device_id=xpeer, device_id_type=MESH,
            )
            rd.wait_recv()

        def fwd(p, c, sems, k, peer):
            return pltpu.make_async_remote_copy(
                src_ref=rbuf.at[p, c], dst_ref=rbuf.at[p, c],
                send_sem=sems.at[k], recv_sem=recv_sems.at[p, c],
                device_id=peer, device_id_type=MESH,
            )

        order = [pm, pyn, pzn, pd]
        rbase = (1 - mx) * v_loc
        yk = 0
        zk = 0
        for j in range(PARTS):
            p = order[j]
            for c in range(SUB):
                recv_wait(p, c)
                if j == 0:
                    fwd(p, c, ys_sems, yk, ypeer).start()
                    yk += 1
                    fwd(p, c, zs_sems, zk, zpeer).start()
                    zk += 1
                elif j == 1 and XD + yd <= c:
                    fwd(p, c, zs_sems, zk, zpeer).start()
                    zk += 1
                elif j == 2 and XD <= c < XD + yd:
                    fwd(p, c, ys_sems, yk, ypeer).start()
                    yk += 1
                e = jnp.exp(rbuf[p, c])
                s = s + jnp.sum(e, axis=-1, keepdims=True)
                out_ref[:, pl.ds(rbase + p * part_cols + c * ch, ch)] = e

        out_ref[:, :] = out_ref[:, :] * (1.0 / s)

        for c in range(SUB):
            r = pltpu.make_async_remote_copy(
                src_ref=lbuf.at[pm, c], dst_ref=rbuf.at[pm, c],
                send_sem=xs_sems.at[c], recv_sem=recv_sems.at[pm, c],
                device_id=xpeer, device_id_type=MESH,
            )
            r.wait_send()
        for c in range(XD):
            r = pltpu.make_async_remote_copy(
                src_ref=lbuf.at[pd, c], dst_ref=rbuf.at[pd, c],
                send_sem=xd_sems.at[c], recv_sem=recv_sems.at[pd, c],
                device_id=xpeer, device_id_type=MESH,
            )
            r.wait_send()
        yk = 0
        zk = 0
        for c in range(SUB):
            fwd(pm, c, ys_sems, yk, ypeer).wait_send()
            yk += 1
            fwd(pm, c, zs_sems, zk, zpeer).wait_send()
            zk += 1
        for c in range(XD + yd, SUB):
            fwd(pyn, c, zs_sems, zk, zpeer).wait_send()
            zk += 1
        for c in range(XD, XD + yd):
            fwd(pzn, c, ys_sems, yk, ypeer).wait_send()
            yk += 1

    return pl.pallas_call(
        body,
        out_shape=jax.ShapeDtypeStruct((t, v_glob), jnp.float32),
        in_specs=[
            pl.BlockSpec(memory_space=pltpu.VMEM),
            pl.BlockSpec(memory_space=pltpu.VMEM),
        ],
        out_specs=pl.BlockSpec(memory_space=pltpu.VMEM),
        scratch_shapes=[
            pltpu.VMEM((PARTS, SUB, t, ch), jnp.float32),
            pltpu.VMEM((PARTS, SUB, t, ch), jnp.float32),
            pltpu.SemaphoreType.DMA((SUB,)),
            pltpu.SemaphoreType.DMA((XD,)),
            pltpu.SemaphoreType.DMA((SUB + yd,)),
            pltpu.SemaphoreType.DMA((SUB + zd,)),
            pltpu.SemaphoreType.DMA((PARTS, SUB)),
        ],
        compiler_params=pltpu.CompilerParams(collective_id=0),
    )(x, W)


# device time: 35641 ns/iter; 1.7190x vs baseline; 1.0485x over previous
import jax
import jax.numpy as jnp
from jax import lax
from jax.experimental import pallas as pl
from jax.experimental.pallas import tpu as pltpu

PARTS = 4
SUB = 8
XD = 3
MESH = pl.DeviceIdType.MESH


def kernel(x, W):
    t, d = x.shape
    _, v_loc = W.shape
    v_glob = 2 * v_loc
    part_cols = v_loc // PARTS
    ch = part_cols // SUB
    yd = (SUB - XD + 1) // 2
    zd = SUB - XD - yd

    def body(x_ref, w_ref, out_ref, lbuf, rbuf, xs_sems, xd_sems, ys_sems,
             zs_sems, recv_sems):
        mx = lax.axis_index("x")
        my = lax.axis_index("y")
        mz = lax.axis_index("z")
        xpeer = (1 - mx, my, mz)
        ypeer = (mx, 1 - my, mz)
        zpeer = (mx, my, 1 - mz)
        pm = 2 * my + mz
        pyn = 2 * (1 - my) + mz
        pzn = 2 * my + (1 - mz)
        pd = 2 * (1 - my) + (1 - mz)

        barrier = pltpu.get_barrier_semaphore()
        for nbr in (xpeer, ypeer, zpeer):
            pl.semaphore_signal(barrier, inc=1, device_id=nbr,
                                device_id_type=MESH)
        pl.semaphore_wait(barrier, 3)

        xv = x_ref[:, :]
        for j in range(PARTS):
            p = lax.rem(pm + j, PARTS)
            for c in range(SUB):
                col = p * part_cols + c * ch
                lbuf[p, c] = jnp.dot(
                    xv, w_ref[:, pl.ds(col, ch)],
                    preferred_element_type=jnp.float32,
                )
                if j == 0:
                    r = pltpu.make_async_remote_copy(
                        src_ref=lbuf.at[p, c], dst_ref=rbuf.at[p, c],
                        send_sem=xs_sems.at[c], recv_sem=recv_sems.at[p, c],
                        device_id=xpeer, device_id_type=MESH,
                    )
                    r.start()

        for c in range(XD):
            r = pltpu.make_async_remote_copy(
                src_ref=lbuf.at[pd, c], dst_ref=rbuf.at[pd, c],
                send_sem=xd_sems.at[c], recv_sem=recv_sems.at[pd, c],
                device_id=xpeer, device_id_type=MESH,
            )
            r.start()

        s = jnp.zeros((t, 1), jnp.float32)
        lbase = mx * v_loc
        for p in range(PARTS):
            for c in range(SUB):
                e = jnp.exp(lbuf[p, c])
                s = s + jnp.sum(e, axis=-1, keepdims=True)
                out_ref[:, pl.ds(lbase + p * part_cols + c * ch, ch)] = e

        def recv_wait(p, c):
            rd = pltpu.make_async_remote_copy(
                src_ref=rbuf.at[p, c], dst_ref=rbuf.at[p, c],
                send_sem=xs_sems.at[0],
                recv_sem=recv_sems.at[p, c],
                device_id=xpeer, device_id_type=MESH,
            )
            rd.wait_recv()

        def fwd(p, c, sems, k, peer):
            return pltpu.make_async_remote_copy(
                src_ref=rbuf.at[p, c], dst_ref=rbuf.at[p, c],
                send_sem=sems.at[k], recv_sem=recv_sems.at[p, c],
                device_id=peer, device_id_type=MESH,
            )

        rbase = (1 - mx) * v_loc
        yk = 0
        zk = 0

        def consume(p, c):
            e = jnp.exp(rbuf[p, c])
            out_ref[:, pl.ds(rbase + p * part_cols + c * ch, ch)] = e
            return jnp.sum(e, axis=-1, keepdims=True)

        for c in range(SUB):
            recv_wait(pm, c)
            fwd(pm, c, ys_sems, yk, ypeer).start()
            yk += 1
            fwd(pm, c, zs_sems, zk, zpeer).start()
            zk += 1
            s = s + consume(pm, c)
        for c in range(SUB):
            recv_wait(pzn, c)
            if XD <= c < XD + yd:
                fwd(pzn, c, ys_sems, yk, ypeer).start()
                yk += 1
            recv_wait(pyn, c)
            if XD + yd <= c:
                fwd(pyn, c, zs_sems, zk, zpeer).start()
                zk += 1
            s = s + consume(pzn, c)
            s = s + consume(pyn, c)
        for c in range(SUB):
            recv_wait(pd, c)
            s = s + consume(pd, c)

        out_ref[:, :] = out_ref[:, :] * (1.0 / s)

        for c in range(SUB):
            r = pltpu.make_async_remote_copy(
                src_ref=lbuf.at[pm, c], dst_ref=rbuf.at[pm, c],
                send_sem=xs_sems.at[c], recv_sem=recv_sems.at[pm, c],
                device_id=xpeer, device_id_type=MESH,
            )
            r.wait_send()
        for c in range(XD):
            r = pltpu.make_async_remote_copy(
                src_ref=lbuf.at[pd, c], dst_ref=rbuf.at[pd, c],
                send_sem=xd_sems.at[c], recv_sem=recv_sems.at[pd, c],
                device_id=xpeer, device_id_type=MESH,
            )
            r.wait_send()
        yk = 0
        zk = 0
        for c in range(SUB):
            fwd(pm, c, ys_sems, yk, ypeer).wait_send()
            yk += 1
            fwd(pm, c, zs_sems, zk, zpeer).wait_send()
            zk += 1
        for c in range(XD + yd, SUB):
            fwd(pyn, c, zs_sems, zk, zpeer).wait_send()
            zk += 1
        for c in range(XD, XD + yd):
            fwd(pzn, c, ys_sems, yk, ypeer).wait_send()
            yk += 1

    return pl.pallas_call(
        body,
        out_shape=jax.ShapeDtypeStruct((t, v_glob), jnp.float32),
        in_specs=[
            pl.BlockSpec(memory_space=pltpu.VMEM),
            pl.BlockSpec(memory_space=pltpu.VMEM),
        ],
        out_specs=pl.BlockSpec(memory_space=pltpu.VMEM),
        scratch_shapes=[
            pltpu.VMEM((PARTS, SUB, t, ch), jnp.float32),
            pltpu.VMEM((PARTS, SUB, t, ch), jnp.float32),
            pltpu.SemaphoreType.DMA((SUB,)),
            pltpu.SemaphoreType.DMA((XD,)),
            pltpu.SemaphoreType.DMA((SUB + yd,)),
            pltpu.SemaphoreType.DMA((SUB + zd,)),
            pltpu.SemaphoreType.DMA((PARTS, SUB)),
        ],
        compiler_params=pltpu.CompilerParams(collective_id=0),
    )(x, W)


# device time: 34750 ns/iter; 1.7631x vs baseline; 1.0256x over previous
import jax
import jax.numpy as jnp
from jax import lax
from jax.experimental import pallas as pl
from jax.experimental.pallas import tpu as pltpu

PARTS = 4
SUB = 8
XD = 4
MESH = pl.DeviceIdType.MESH


def kernel(x, W):
    t, d = x.shape
    _, v_loc = W.shape
    v_glob = 2 * v_loc
    part_cols = v_loc // PARTS
    ch = part_cols // SUB
    yd = (SUB - XD + 1) // 2
    zd = SUB - XD - yd

    def body(x_ref, w_ref, out_ref, lbuf, rbuf, xs_sems, xd_sems, ys_sems,
             zs_sems, recv_sems):
        mx = lax.axis_index("x")
        my = lax.axis_index("y")
        mz = lax.axis_index("z")
        xpeer = (1 - mx, my, mz)
        ypeer = (mx, 1 - my, mz)
        zpeer = (mx, my, 1 - mz)
        pm = 2 * my + mz
        pyn = 2 * (1 - my) + mz
        pzn = 2 * my + (1 - mz)
        pd = 2 * (1 - my) + (1 - mz)

        barrier = pltpu.get_barrier_semaphore()
        for nbr in (xpeer, ypeer, zpeer):
            pl.semaphore_signal(barrier, inc=1, device_id=nbr,
                                device_id_type=MESH)
        pl.semaphore_wait(barrier, 3)

        xv = x_ref[:, :]
        for j in range(PARTS):
            p = lax.rem(pm + j, PARTS)
            for c in range(SUB):
                col = p * part_cols + c * ch
                lbuf[p, c] = jnp.dot(
                    xv, w_ref[:, pl.ds(col, ch)],
                    preferred_element_type=jnp.float32,
                )
                if j == 0:
                    r = pltpu.make_async_remote_copy(
                        src_ref=lbuf.at[p, c], dst_ref=rbuf.at[p, c],
                        send_sem=xs_sems.at[c], recv_sem=recv_sems.at[p, c],
                        device_id=xpeer, device_id_type=MESH,
                    )
                    r.start()

        for c in range(XD):
            r = pltpu.make_async_remote_copy(
                src_ref=lbuf.at[pd, c], dst_ref=rbuf.at[pd, c],
                send_sem=xd_sems.at[c], recv_sem=recv_sems.at[pd, c],
                device_id=xpeer, device_id_type=MESH,
            )
            r.start()

        s = jnp.zeros((t, 1), jnp.float32)
        lbase = mx * v_loc
        for p in range(PARTS):
            for c in range(SUB):
                e = jnp.exp(lbuf[p, c])
                s = s + jnp.sum(e, axis=-1, keepdims=True)
                out_ref[:, pl.ds(lbase + p * part_cols + c * ch, ch)] = e

        def recv_wait(p, c):
            rd = pltpu.make_async_remote_copy(
                src_ref=rbuf.at[p, c], dst_ref=rbuf.at[p, c],
                send_sem=xs_sems.at[0],
                recv_sem=recv_sems.at[p, c],
                device_id=xpeer, device_id_type=MESH,
            )
            rd.wait_recv()

        def fwd(p, c, sems, k, peer):
            return pltpu.make_async_remote_copy(
                src_ref=rbuf.at[p, c], dst_ref=rbuf.at[p, c],
                send_sem=sems.at[k], recv_sem=recv_sems.at[p, c],
                device_id=peer, device_id_type=MESH,
            )

        rbase = (1 - mx) * v_loc
        yk = 0
        zk = 0

        def consume(p, c):
            e = jnp.exp(rbuf[p, c])
            out_ref[:, pl.ds(rbase + p * part_cols + c * ch, ch)] = e
            return jnp.sum(e, axis=-1, keepdims=True)

        for c in range(SUB):
            recv_wait(pm, c)
            fwd(pm, c, ys_sems, yk, ypeer).start()
            yk += 1
            fwd(pm, c, zs_sems, zk, zpeer).start()
            zk += 1
            s = s + consume(pm, c)
        for c in range(SUB):
            recv_wait(pzn, c)
            if XD <= c < XD + yd:
                fwd(pzn, c, ys_sems, yk, ypeer).start()
                yk += 1
            recv_wait(pyn, c)
            if XD + yd <= c:
                fwd(pyn, c, zs_sems, zk, zpeer).start()
                zk += 1
            s = s + consume(pzn, c)
            s = s + consume(pyn, c)
        for c in range(SUB):
            recv_wait(pd, c)
            s = s + consume(pd, c)

        out_ref[:, :] = out_ref[:, :] * (1.0 / s)

        for c in range(SUB):
            r = pltpu.make_async_remote_copy(
                src_ref=lbuf.at[pm, c], dst_ref=rbuf.at[pm, c],
                send_sem=xs_sems.at[c], recv_sem=recv_sems.at[pm, c],
                device_id=xpeer, device_id_type=MESH,
            )
            r.wait_send()
        for c in range(XD):
            r = pltpu.make_async_remote_copy(
                src_ref=lbuf.at[pd, c], dst_ref=rbuf.at[pd, c],
                send_sem=xd_sems.at[c], recv_sem=recv_sems.at[pd, c],
                device_id=xpeer, device_id_type=MESH,
            )
            r.wait_send()
        yk = 0
        zk = 0
        for c in range(SUB):
            fwd(pm, c, ys_sems, yk, ypeer).wait_send()
            yk += 1
            fwd(pm, c, zs_sems, zk, zpeer).wait_send()
            zk += 1
        for c in range(XD + yd, SUB):
            fwd(pyn, c, zs_sems, zk, zpeer).wait_send()
            zk += 1
        for c in range(XD, XD + yd):
            fwd(pzn, c, ys_sems, yk, ypeer).wait_send()
            yk += 1

    return pl.pallas_call(
        body,
        out_shape=jax.ShapeDtypeStruct((t, v_glob), jnp.float32),
        in_specs=[
            pl.BlockSpec(memory_space=pltpu.VMEM),
            pl.BlockSpec(memory_space=pltpu.VMEM),
        ],
        out_specs=pl.BlockSpec(memory_space=pltpu.VMEM),
        scratch_shapes=[
            pltpu.VMEM((PARTS, SUB, t, ch), jnp.float32),
            pltpu.VMEM((PARTS, SUB, t, ch), jnp.float32),
            pltpu.SemaphoreType.DMA((SUB,)),
            pltpu.SemaphoreType.DMA((XD,)),
            pltpu.SemaphoreType.DMA((SUB + yd,)),
            pltpu.SemaphoreType.DMA((SUB + zd,)),
            pltpu.SemaphoreType.DMA((PARTS, SUB)),
        ],
        compiler_params=pltpu.CompilerParams(collective_id=0),
    )(x, W)
